# Initial kernel scaffold; baseline (speedup 1.0000x reference)
import functools

import jax
import jax.numpy as jnp
from jax import lax
from jax.experimental import pallas as pl
from jax.experimental.pallas import tpu as pltpu
from jax.experimental.pallas import tpu_sc as plsc

N = 10000
E = 320000
D = 128

NC = 2
NS = 16
NW = NC * NS
NODES_PER_W = 313
N_PAD = NODES_PER_W * NW

EB = 3200
NBLK = E // EB
CHUNKS = EB // 16
GB = 128

NEG_INF = float("-inf")


ROW_BLK = 400


def _pool_body(feat_ref, w_ref, b_ref, h_ref):
    x = feat_ref[...]
    w = w_ref[...]
    y = lax.dot_general(x, w, (((1,), (1,)), ((), ())),
                        preferred_element_type=jnp.float32)
    h_ref[...] = jnp.maximum(y + b_ref[...], 0.0)


def _out_body(feat_ref, neigh_ref, ws_ref, bs_ref, wn_ref, bn_ref, o_ref):
    a = lax.dot_general(feat_ref[...], ws_ref[...], (((1,), (1,)), ((), ())),
                        preferred_element_type=jnp.float32)
    b = lax.dot_general(neigh_ref[...], wn_ref[...], (((1,), (1,)), ((), ())),
                        preferred_element_type=jnp.float32)
    o_ref[...] = a + bs_ref[...] + b + bn_ref[...]


def _tc_pool(feat, W_pool, b_pool):
    return pl.pallas_call(
        _pool_body,
        grid=(N // ROW_BLK,),
        in_specs=[
            pl.BlockSpec((ROW_BLK, D), lambda i: (i, 0)),
            pl.BlockSpec((D, D), lambda i: (0, 0)),
            pl.BlockSpec((1, D), lambda i: (0, 0)),
        ],
        out_specs=pl.BlockSpec((ROW_BLK, D), lambda i: (i, 0)),
        out_shape=jax.ShapeDtypeStruct((N, D), jnp.float32),
    )(feat, W_pool, b_pool.reshape(1, D))


def _tc_out(feat, neigh, W_self, b_self, W_neigh, b_neigh):
    return pl.pallas_call(
        _out_body,
        grid=(N // ROW_BLK,),
        in_specs=[
            pl.BlockSpec((ROW_BLK, D), lambda i: (i, 0)),
            pl.BlockSpec((ROW_BLK, D), lambda i: (i, 0)),
            pl.BlockSpec((D, D), lambda i: (0, 0)),
            pl.BlockSpec((1, D), lambda i: (0, 0)),
            pl.BlockSpec((D, D), lambda i: (0, 0)),
            pl.BlockSpec((1, D), lambda i: (0, 0)),
        ],
        out_specs=pl.BlockSpec((ROW_BLK, D), lambda i: (i, 0)),
        out_shape=jax.ShapeDtypeStruct((N, D), jnp.float32),
    )(feat, neigh, W_self, b_self.reshape(1, D), W_neigh, b_neigh.reshape(1, D))


def _sc_segmax(h, src, dst, weight):
    mesh = plsc.VectorSubcoreMesh(core_axis_name="c", subcore_axis_name="s")

    @functools.partial(
        pl.kernel,
        mesh=mesh,
        out_type=jax.ShapeDtypeStruct((N_PAD, D), jnp.float32),
        scratch_types=[
            pltpu.VMEM((NODES_PER_W, D), jnp.float32),
            pltpu.VMEM((EB,), jnp.int32),
            pltpu.VMEM((EB,), jnp.int32),
            pltpu.VMEM((EB,), jnp.float32),
            pltpu.VMEM((EB + GB,), jnp.int32),
            pltpu.VMEM((EB + GB,), jnp.int32),
            pltpu.VMEM((EB + GB,), jnp.float32),
            pltpu.VMEM((GB, D), jnp.float32),
            pltpu.SemaphoreType.DMA,
            pltpu.SemaphoreType.DMA,
        ],
    )
    def k(h_hbm, src_hbm, dst_hbm, w_hbm, out_hbm,
          acc, eb_src, eb_dst, eb_w, c_src, c_dst, c_w, rows, sem, gsem):
        wid = lax.axis_index("s") * NC + lax.axis_index("c")
        lo = wid * NODES_PER_W

        neg = jnp.full((16,), NEG_INF, dtype=jnp.float32)

        @pl.loop(0, NODES_PER_W)
        def _(i):
            for c in range(D // 16):
                acc[i, pl.ds(c * 16, 16)] = neg

        def do_block(b, _):
            base = b * EB
            cp1 = pltpu.async_copy(src_hbm.at[pl.ds(base, EB)], eb_src, sem)
            cp2 = pltpu.async_copy(dst_hbm.at[pl.ds(base, EB)], eb_dst, sem)
            cp3 = pltpu.async_copy(w_hbm.at[pl.ds(base, EB)], eb_w, sem)
            cp1.wait()
            cp2.wait()
            cp3.wait()

            def scan_chunk(ci, off):
                d = eb_dst[pl.ds(ci * 16, 16)]
                dl = d - lo
                msk = (dl >= 0) & (dl < NODES_PER_W)
                plsc.store_compressed(c_src.at[pl.ds(off, 16)],
                                      eb_src[pl.ds(ci * 16, 16)], msk)
                plsc.store_compressed(c_dst.at[pl.ds(off, 16)], dl, msk)
                plsc.store_compressed(c_w.at[pl.ds(off, 16)],
                                      eb_w[pl.ds(ci * 16, 16)], msk)
                cnt = jnp.max(plsc.all_reduce_population_count(msk))
                return off + cnt

            off = lax.fori_loop(0, CHUNKS, scan_chunk, jnp.int32(0))

            zeros = jnp.zeros((16,), jnp.int32)
            for t in range(GB // 16):
                c_src[pl.ds(off + t * 16, 16)] = zeros

            nbat = (off + GB - 1) // GB

            def do_batch(g, _):
                gb = g * GB
                pltpu.async_copy(h_hbm.at[c_src.at[pl.ds(gb, GB)]], rows,
                                 gsem).wait()
                cnt_g = jnp.minimum(off - gb, GB)

                def do_edge(j, _):
                    dj = c_dst[gb + j]
                    wj = c_w[gb + j]
                    ws = jnp.full((16,), wj, dtype=jnp.float32)
                    for c in range(D // 16):
                        rv = rows[j, pl.ds(c * 16, 16)]
                        av = acc[dj, pl.ds(c * 16, 16)]
                        acc[dj, pl.ds(c * 16, 16)] = jnp.maximum(av, rv * ws)
                    return 0

                lax.fori_loop(0, cnt_g, do_edge, 0)
                return 0

            lax.fori_loop(0, nbat, do_batch, 0)
            return 0

        lax.fori_loop(0, NBLK, do_block, 0)

        zero16 = jnp.zeros((16,), jnp.float32)

        @pl.loop(0, NODES_PER_W)
        def _(i):
            for c in range(D // 16):
                v = acc[i, pl.ds(c * 16, 16)]
                acc[i, pl.ds(c * 16, 16)] = jnp.where(v == NEG_INF, zero16, v)

        pltpu.sync_copy(acc, out_hbm.at[pl.ds(lo, NODES_PER_W)])

    return k(h, src, dst, weight)


@jax.jit
def kernel(feat, edge_index, weight, W_pool, b_pool, W_self, b_self,
           W_neigh, b_neigh):
    h = _tc_pool(feat, W_pool, b_pool)
    src = edge_index[0]
    dst = edge_index[1]
    w = weight.reshape(E)
    neigh_pad = _sc_segmax(h, src, dst, w)
    neigh = neigh_pad[:N]
    return _tc_out(feat, neigh, W_self, b_self, W_neigh, b_neigh)


# speedup vs baseline: 2.0081x; 2.0081x over previous
import functools

import jax
import jax.numpy as jnp
from jax import lax
from jax.experimental import pallas as pl
from jax.experimental.pallas import tpu as pltpu
from jax.experimental.pallas import tpu_sc as plsc

N = 10000
E = 320000
D = 128

NC = 2
NS = 16
NW = NC * NS
NODES_PER_W = 320
N_PAD = NODES_PER_W * NW

EB = 3200
NBLK = E // EB
CHUNKS = EB // 16
GB = 128

NEG_INF = float("-inf")


ROW_BLK = 400


def _pool_body(feat_ref, w_ref, b_ref, h_ref):
    x = feat_ref[...]
    w = w_ref[...]
    y = lax.dot_general(x, w, (((1,), (1,)), ((), ())),
                        preferred_element_type=jnp.float32)
    h_ref[...] = jnp.maximum(y + b_ref[...], 0.0)


def _out_body(feat_ref, neigh_ref, ws_ref, bs_ref, wn_ref, bn_ref, o_ref):
    a = lax.dot_general(feat_ref[...], ws_ref[...], (((1,), (1,)), ((), ())),
                        preferred_element_type=jnp.float32)
    b = lax.dot_general(neigh_ref[...], wn_ref[...], (((1,), (1,)), ((), ())),
                        preferred_element_type=jnp.float32)
    o_ref[...] = a + bs_ref[...] + b + bn_ref[...]


def _tc_pool(feat, W_pool, b_pool):
    return pl.pallas_call(
        _pool_body,
        grid=(N // ROW_BLK,),
        in_specs=[
            pl.BlockSpec((ROW_BLK, D), lambda i: (i, 0)),
            pl.BlockSpec((D, D), lambda i: (0, 0)),
            pl.BlockSpec((1, D), lambda i: (0, 0)),
        ],
        out_specs=pl.BlockSpec((ROW_BLK, D), lambda i: (i, 0)),
        out_shape=jax.ShapeDtypeStruct((N, D), jnp.float32),
    )(feat, W_pool, b_pool.reshape(1, D))


def _tc_out(feat, neigh, W_self, b_self, W_neigh, b_neigh):
    return pl.pallas_call(
        _out_body,
        grid=(N // ROW_BLK,),
        in_specs=[
            pl.BlockSpec((ROW_BLK, D), lambda i: (i, 0)),
            pl.BlockSpec((ROW_BLK, D), lambda i: (i, 0)),
            pl.BlockSpec((D, D), lambda i: (0, 0)),
            pl.BlockSpec((1, D), lambda i: (0, 0)),
            pl.BlockSpec((D, D), lambda i: (0, 0)),
            pl.BlockSpec((1, D), lambda i: (0, 0)),
        ],
        out_specs=pl.BlockSpec((ROW_BLK, D), lambda i: (i, 0)),
        out_shape=jax.ShapeDtypeStruct((N, D), jnp.float32),
    )(feat, neigh, W_self, b_self.reshape(1, D), W_neigh, b_neigh.reshape(1, D))


def _sc_segmax(h, src, dst, weight):
    mesh = plsc.VectorSubcoreMesh(core_axis_name="c", subcore_axis_name="s")

    CAP = EB + 2 * GB

    @functools.partial(
        pl.kernel,
        mesh=mesh,
        compiler_params=pltpu.CompilerParams(needs_layout_passes=False),
        out_type=jax.ShapeDtypeStruct((N_PAD, D), jnp.float32),
        scratch_types=[
            pltpu.VMEM((NODES_PER_W + 1, D), jnp.float32),
            pltpu.VMEM((EB,), jnp.int32),
            pltpu.VMEM((EB,), jnp.int32),
            pltpu.VMEM((EB,), jnp.float32),
            pltpu.VMEM((CAP,), jnp.int32),
            pltpu.VMEM((CAP,), jnp.int32),
            pltpu.VMEM((CAP,), jnp.float32),
            pltpu.VMEM((GB, D), jnp.float32),
            pltpu.SemaphoreType.DMA,
            pltpu.SemaphoreType.DMA,
        ],
    )
    def k(h_hbm, src_hbm, dst_hbm, w_hbm, out_hbm,
          acc, eb_src, eb_dst, eb_w, c_src, c_dst, c_w, rows, sem, gsem):
        wid = lax.axis_index("s") * NC + lax.axis_index("c")
        lo = wid * NODES_PER_W

        neg = jnp.full((16,), NEG_INF, dtype=jnp.float32)

        @pl.loop(0, NODES_PER_W + 1)
        def _(i):
            for c in range(D // 16):
                acc[i, pl.ds(c * 16, 16)] = neg

        def run_batch(gb):
            pltpu.async_copy(h_hbm.at[c_src.at[pl.ds(gb, GB)]], rows,
                             gsem).wait()

            def grp(g2, _):
                s = gb + g2 * 16
                dvec = c_dst[pl.ds(s, 16)]
                wvec = c_w[pl.ds(s, 16)]
                for t in range(16):
                    dj = dvec[t]
                    ws = jnp.full((16,), wvec[t], dtype=jnp.float32)
                    j = g2 * 16 + t
                    for c in range(D // 16):
                        rv = rows[j, pl.ds(c * 16, 16)]
                        av = acc[dj, pl.ds(c * 16, 16)]
                        acc[dj, pl.ds(c * 16, 16)] = jnp.maximum(av, rv * ws)
                return 0

            lax.fori_loop(0, GB // 16, grp, 0)

        def do_block(b, off):
            base = b * EB
            cp1 = pltpu.async_copy(src_hbm.at[pl.ds(base, EB)], eb_src, sem)
            cp2 = pltpu.async_copy(dst_hbm.at[pl.ds(base, EB)], eb_dst, sem)
            cp3 = pltpu.async_copy(w_hbm.at[pl.ds(base, EB)], eb_w, sem)
            cp1.wait()
            cp2.wait()
            cp3.wait()

            def scan_chunk(ci, o):
                d = eb_dst[pl.ds(ci * 16, 16)]
                dl = d - lo
                msk = (dl >= 0) & (dl < NODES_PER_W)
                plsc.store_compressed(c_src.at[pl.ds(o, 16)],
                                      eb_src[pl.ds(ci * 16, 16)], mask=msk)
                plsc.store_compressed(c_dst.at[pl.ds(o, 16)], dl, mask=msk)
                plsc.store_compressed(c_w.at[pl.ds(o, 16)],
                                      eb_w[pl.ds(ci * 16, 16)], mask=msk)
                cnt = jnp.max(plsc.all_reduce_population_count(msk))
                return o + cnt

            off = lax.fori_loop(0, CHUNKS, scan_chunk, off)

            nb = off // GB

            def do_batch(g, _):
                run_batch(g * GB)
                return 0

            lax.fori_loop(0, nb, do_batch, 0)

            tail = nb * GB
            for t in range(GB // 16):
                c_src[pl.ds(t * 16, 16)] = c_src[pl.ds(tail + t * 16, 16)]
                c_dst[pl.ds(t * 16, 16)] = c_dst[pl.ds(tail + t * 16, 16)]
                c_w[pl.ds(t * 16, 16)] = c_w[pl.ds(tail + t * 16, 16)]
            return off - tail

        off = lax.fori_loop(0, NBLK, do_block, jnp.int32(0))

        zero_i = jnp.zeros((16,), jnp.int32)
        zero_f = jnp.zeros((16,), jnp.float32)
        dummy_d = jnp.full((16,), NODES_PER_W, dtype=jnp.int32)
        for t in range(GB // 16):
            c_src[pl.ds(off + t * 16, 16)] = zero_i
            c_dst[pl.ds(off + t * 16, 16)] = dummy_d
            c_w[pl.ds(off + t * 16, 16)] = zero_f
        run_batch(0)

        zero16 = jnp.zeros((16,), jnp.float32)

        @pl.loop(0, NODES_PER_W)
        def _(i):
            for c in range(D // 16):
                v = acc[i, pl.ds(c * 16, 16)]
                acc[i, pl.ds(c * 16, 16)] = jnp.where(v == NEG_INF, zero16, v)

        pltpu.sync_copy(acc.at[pl.ds(0, NODES_PER_W)],
                        out_hbm.at[pl.ds(lo, NODES_PER_W)])

    return k(h, src, dst, weight)


@jax.jit
def kernel(feat, edge_index, weight, W_pool, b_pool, W_self, b_self,
           W_neigh, b_neigh):
    h = _tc_pool(feat, W_pool, b_pool)
    src = edge_index[0]
    dst = edge_index[1]
    w = weight.reshape(E)
    neigh_pad = _sc_segmax(h, src, dst, w)
    neigh = neigh_pad[:N]
    return _tc_out(feat, neigh, W_self, b_self, W_neigh, b_neigh)
